# baseline (speedup 1.0000x reference)
import functools

import jax
import jax.numpy as jnp
from jax import lax
from jax.experimental import pallas as pl
from jax.experimental.pallas import tpu as pltpu
from jax.experimental.pallas import tpu_sc as plsc

BATCH = 4096
INPUT_DIM = 39
N_CONT = 13
N_CAT = 26
VOCAB = 100000
EMB = 32
LANES = 16

NUM_CORES = 2
NUM_SUBCORES = 16
NUM_WORKERS = NUM_CORES * NUM_SUBCORES
B_PER_W = BATCH // NUM_WORKERS
VREGS_PER_TABLE = B_PER_W // LANES
NBUF = 2

_mesh = plsc.VectorSubcoreMesh(
    core_axis_name="c", subcore_axis_name="s",
    num_cores=NUM_CORES, num_subcores=NUM_SUBCORES,
)


@functools.partial(
    pl.kernel,
    out_type=jax.ShapeDtypeStruct((N_CAT, BATCH, EMB), jnp.float32),
    mesh=_mesh,
    compiler_params=pltpu.CompilerParams(use_tc_tiling_on_sc=False),
    scratch_types=(
        [pltpu.VMEM((N_CAT, B_PER_W), jnp.int32)]
        + [pltpu.VMEM((B_PER_W, EMB), jnp.float32) for _ in range(NBUF)]
        + [pltpu.SemaphoreType.DMA for _ in range(1 + NBUF)]
    ),
)
def _emb_kernel(tables_hbm, gidx_hbm, out_hbm, gidx_v, *bufs_and_sems):
    bufs = bufs_and_sems[:NBUF]
    gsem = bufs_and_sems[NBUF]
    wsems = bufs_and_sems[NBUF + 1:]

    wid = lax.axis_index("s") * NUM_CORES + lax.axis_index("c")
    base_b = wid * B_PER_W

    pltpu.sync_copy(gidx_hbm.at[:, pl.ds(base_b, B_PER_W)], gidx_v)

    writes = [None] * N_CAT
    for j in range(N_CAT):
        slot = j % NBUF
        if j >= NBUF:
            writes[j - NBUF].wait()
        gathers = []
        for h in range(VREGS_PER_TABLE):
            idx16 = gidx_v[j, pl.ds(h * LANES, LANES)]
            gathers.append(pltpu.async_copy(
                tables_hbm.at[idx16],
                bufs[slot].at[pl.ds(h * LANES, LANES), :],
                gsem))
        for g in gathers:
            g.wait()
        writes[j] = pltpu.async_copy(
            bufs[slot],
            out_hbm.at[j, pl.ds(base_b, B_PER_W), :],
            wsems[slot])
    for j in range(N_CAT - NBUF, N_CAT):
        writes[j].wait()


def kernel(x, tables):
    gidx = x[:, N_CONT:].astype(jnp.int32).T + (
        jnp.arange(N_CAT, dtype=jnp.int32) * VOCAB
    )[:, None]
    tables_flat = tables.reshape(N_CAT * VOCAB, EMB)
    emb = _emb_kernel(tables_flat, gidx)
    return jnp.concatenate(
        [x[:, :N_CONT]] + [emb[j] for j in range(N_CAT)], axis=1)


# speedup vs baseline: 1.0606x; 1.0606x over previous
import functools

import jax
import jax.numpy as jnp
from jax import lax
from jax.experimental import pallas as pl
from jax.experimental.pallas import tpu as pltpu
from jax.experimental.pallas import tpu_sc as plsc

BATCH = 4096
INPUT_DIM = 39
N_CONT = 13
N_CAT = 26
VOCAB = 100000
EMB = 32
LANES = 16
PAD = 3
PAD_OUT = PAD + N_CONT + N_CAT * EMB
CONT_BLK = PAD + N_CONT

NUM_CORES = 2
NUM_SUBCORES = 16
NUM_WORKERS = NUM_CORES * NUM_SUBCORES
B_PER_W = BATCH // NUM_WORKERS
VREGS_PER_TABLE = B_PER_W // LANES
NBUF = 2

_mesh = plsc.VectorSubcoreMesh(
    core_axis_name="c", subcore_axis_name="s",
    num_cores=NUM_CORES, num_subcores=NUM_SUBCORES,
)


@functools.partial(
    pl.kernel,
    out_type=jax.ShapeDtypeStruct((BATCH, PAD_OUT), jnp.float32),
    mesh=_mesh,
    compiler_params=pltpu.CompilerParams(use_tc_tiling_on_sc=False),
    scratch_types=(
        [
            pltpu.VMEM((N_CAT, B_PER_W), jnp.int32),
            pltpu.VMEM((B_PER_W, CONT_BLK), jnp.float32),
        ]
        + [pltpu.VMEM((B_PER_W, EMB), jnp.float32) for _ in range(NBUF)]
        + [pltpu.SemaphoreType.DMA for _ in range(2 + NBUF)]
    ),
)
def _emb_kernel(tables_hbm, gidx_hbm, xc_hbm, out_hbm,
                gidx_v, cont_v, *bufs_and_sems):
    bufs = bufs_and_sems[:NBUF]
    gsem, csem = bufs_and_sems[NBUF], bufs_and_sems[NBUF + 1]
    wsems = bufs_and_sems[NBUF + 2:]

    wid = lax.axis_index("s") * NUM_CORES + lax.axis_index("c")
    base_b = wid * B_PER_W

    pltpu.sync_copy(gidx_hbm.at[:, pl.ds(base_b, B_PER_W)], gidx_v)

    pltpu.sync_copy(xc_hbm.at[pl.ds(base_b, B_PER_W)], cont_v)
    cont = pltpu.async_copy(
        cont_v, out_hbm.at[pl.ds(base_b, B_PER_W), pl.ds(0, CONT_BLK)], csem)

    writes = [None] * N_CAT
    for j in range(N_CAT):
        slot = j % NBUF
        if j >= NBUF:
            writes[j - NBUF].wait()
        gathers = []
        for h in range(VREGS_PER_TABLE):
            idx16 = gidx_v[j, pl.ds(h * LANES, LANES)]
            gathers.append(pltpu.async_copy(
                tables_hbm.at[idx16],
                bufs[slot].at[pl.ds(h * LANES, LANES), :],
                gsem))
        for g in gathers:
            g.wait()
        writes[j] = pltpu.async_copy(
            bufs[slot],
            out_hbm.at[pl.ds(base_b, B_PER_W),
                       pl.ds(CONT_BLK + j * EMB, EMB)],
            wsems[slot])
    for j in range(N_CAT - NBUF, N_CAT):
        writes[j].wait()
    cont.wait()


def kernel(x, tables):
    gidx = x[:, N_CONT:].astype(jnp.int32).T + (
        jnp.arange(N_CAT, dtype=jnp.int32) * VOCAB
    )[:, None]
    tables_flat = tables.reshape(N_CAT * VOCAB, EMB)
    xc = jnp.pad(x[:, :N_CONT], ((0, 0), (PAD, 0)))
    padded = _emb_kernel(tables_flat, gidx, xc)
    return padded[:, PAD:]
